# baseline (speedup 1.0000x reference)
import jax
import jax.numpy as jnp
from jax import lax
from jax.experimental import pallas as pl
from jax.experimental.pallas import tpu as pltpu
from jax.experimental.pallas import tpu_sc as plsc

B, H, W, P = 4096, 24, 24, 10
HW = H * W
EC = 5
C = EC + 1 + P + 10
NC, NS, L = 2, 16, 16
NW = NC * NS
BAND = 8
NBAND = W // BAND
NVREG = B // L


def _sc_body(tt_hbm, scal_hbm, tbl_hbm, g_hbm, out_hbm,
             tt_v, bufA, bufB, tbl_v, sem_g, sem_b, sem_e):
    wid = lax.axis_index("s") * NC + lax.axis_index("c")

    pltpu.sync_copy(tbl_hbm, tbl_v)
    v0 = tbl_v[pl.ds(0, L)]
    v1 = tbl_v[pl.ds(4, L)]
    tvec = [[jnp.full((L,), v0[r * EC + c], jnp.float32) if r * EC + c < L
             else jnp.full((L,), v1[r * EC + c - 4], jnp.float32)
             for c in range(EC)] for r in range(4)]

    grid_jobs = []
    for g in range(10):
        m = lax.rem(wid - 7 * g + 7 * 10 * 21, 21)
        for extra in (0, 21):
            h = m + extra
            cond = jnp.logical_and(wid < 21, h < H)
            src = g_hbm[g].at[pl.ds(h * W, W)]
            dst = out_hbm.at[h, EC + 1 + P + g]
            grid_jobs.append((cond, src, dst))
    for cond, src, dst in grid_jobs:
        @pl.when(cond)
        def _():
            pltpu.async_copy(src, dst, sem_g)

    def emb_task(t):
        h = t // NBAND
        r = t - h * NBAND
        pltpu.sync_copy(tt_hbm.at[pl.ds(h * W + r * BAND, BAND)], tt_v)
        for c in range(EC):
            buf = bufA if c % 2 == 0 else bufB
            if c >= 2:
                pltpu.make_async_copy(
                    buf, out_hbm.at[h, c - 2, pl.ds(r * BAND, BAND)],
                    sem_e).wait()
            t0, t1, t2, t3 = (tvec[0][c], tvec[1][c], tvec[2][c], tvec[3][c])
            for row in range(BAND):
                def lp(j, carry, row=row, t0=t0, t1=t1, t2=t2, t3=t3):
                    tt = tt_v[row, pl.ds(j * L, L)]
                    val = jnp.where(tt < 2, jnp.where(tt == 0, t0, t1),
                                    jnp.where(tt == 2, t2, t3))
                    buf[row, pl.ds(j * L, L)] = val
                    return carry
                lax.fori_loop(0, NVREG, lp, 0)
            pltpu.async_copy(buf, out_hbm.at[h, c, pl.ds(r * BAND, BAND)],
                             sem_e)
        for c in (EC - 2, EC - 1):
            buf = bufA if c % 2 == 0 else bufB
            pltpu.make_async_copy(
                buf, out_hbm.at[h, c, pl.ds(r * BAND, BAND)], sem_e).wait()

    for k in range(3):
        t = wid + k * NW
        if k < 2:
            emb_task(t)
        else:
            @pl.when(wid < 72 - 2 * NW)
            def _():
                emb_task(t)

    j = wid - 21
    on_bcast = jnp.logical_and(wid >= 21, wid < NW)
    bc_jobs = []

    @pl.when(on_bcast)
    def _():
        band = j // BAND
        j8 = j - band * BAND
        pltpu.sync_copy(scal_hbm.at[pl.ds(band * BAND, BAND)], bufA)
        def bt_loop(k, carry):
            v = bufA[j8, pl.ds(k * L, L)]
            for ws in range(BAND):
                bufB[ws, pl.ds(k * L, L)] = v
            return carry
        lax.fori_loop(0, NVREG, bt_loop, 0)

    for h in range(H):
        for r in range(NBAND):
            cond = on_bcast
            dst = out_hbm.at[h, EC + j, pl.ds(r * BAND, BAND)]
            bc_jobs.append((cond, dst))
    for cond, dst in bc_jobs:
        @pl.when(cond)
        def _():
            pltpu.async_copy(bufB, dst, sem_b)

    for cond, src, dst in grid_jobs:
        @pl.when(cond)
        def _():
            pltpu.make_async_copy(src, dst, sem_g).wait()
    for cond, dst in bc_jobs:
        @pl.when(cond)
        def _():
            pltpu.make_async_copy(bufB, dst, sem_b).wait()


@jax.jit
def _encode_sc(tt, scal, tbl, grids):
    mesh = plsc.VectorSubcoreMesh(core_axis_name="c", subcore_axis_name="s")

    def body(tt_hbm, scal_hbm, tbl_hbm, g0, g1, g2, g3, g4, g5, g6, g7, g8,
             g9, out_hbm, tt_v, bufA, bufB, tbl_v, sem_g, sem_b, sem_e):
        _sc_body(tt_hbm, scal_hbm, tbl_hbm,
                 (g0, g1, g2, g3, g4, g5, g6, g7, g8, g9), out_hbm,
                 tt_v, bufA, bufB, tbl_v, sem_g, sem_b, sem_e)

    run = pl.kernel(
        body,
        out_type=jax.ShapeDtypeStruct((H, C, W, B), jnp.float32),
        mesh=mesh,
        scratch_types=[
            pltpu.VMEM((BAND, B), jnp.int32),
            pltpu.VMEM((BAND, B), jnp.float32),
            pltpu.VMEM((BAND, B), jnp.float32),
            pltpu.VMEM((20,), jnp.float32),
            pltpu.SemaphoreType.DMA,
            pltpu.SemaphoreType.DMA,
            pltpu.SemaphoreType.DMA,
        ],
        compiler_params=pltpu.CompilerParams(needs_layout_passes=False),
    )
    return run(tt, scal, tbl, *grids)


def kernel_sc(tile_type, normalized_steps, param_list, grids, embed_table):
    tt = jnp.transpose(tile_type.astype(jnp.int32), (1, 2, 0)).reshape(HW, B)
    par_t = jnp.transpose(param_list, (1, 0))
    scal = jnp.concatenate(
        [normalized_steps.astype(jnp.float32).reshape(1, B), par_t,
         jnp.zeros((16 - 1 - P, B), jnp.float32)], axis=0)
    tbl = embed_table.reshape(4 * EC)
    gr = tuple(jnp.transpose(g, (1, 2, 0)).reshape(HW, B) for g in grids)
    out = _encode_sc(tt, scal, tbl, gr)
    return jnp.transpose(out, (3, 0, 2, 1))


def kernel(tile_type, normalized_steps, param_list,
           sensor_mask, normalized_unit_counts, normalized_unit_counts_opp,
           normalized_unit_energys_max_grid, normalized_unit_energys_max_grid_opp,
           grid_probability_of_being_an_energy_point_based_on_no_reward,
           grid_max_probability_of_being_an_energy_point_based_on_positive_rewards,
           grid_avg_probability_of_being_an_energy_point_based_on_positive_rewards,
           grid_probability_of_being_energy_point_based_on_relic_positions,
           value_of_sapping_grid, embed_table):
    grids = (sensor_mask, normalized_unit_counts, normalized_unit_counts_opp,
             normalized_unit_energys_max_grid, normalized_unit_energys_max_grid_opp,
             grid_probability_of_being_an_energy_point_based_on_no_reward,
             grid_max_probability_of_being_an_energy_point_based_on_positive_rewards,
             grid_avg_probability_of_being_an_energy_point_based_on_positive_rewards,
             grid_probability_of_being_energy_point_based_on_relic_positions,
             value_of_sapping_grid)
    return kernel_sc(tile_type, normalized_steps, param_list, grids, embed_table)


# speedup vs baseline: 17.4538x; 17.4538x over previous
import jax
import jax.numpy as jnp
from jax import lax
from jax.experimental import pallas as pl
from jax.experimental.pallas import tpu as pltpu
from jax.experimental.pallas import tpu_sc as plsc

B, H, W, P = 4096, 24, 24, 10
HW = H * W
EC = 5
C = EC + 1 + P + 10
NSC = 5
CTC = C - NSC
NC, NS, L = 2, 16, 16
NW = NC * NS
BAND = 8
NBAND = W // BAND
GB = H * NBAND
BB = 1024
NBC = B // BB


def _wait_band(buf, out_hbm, sem):
    pltpu.make_async_copy(buf, out_hbm.at[0, 0, pl.ds(0, BAND)], sem).wait()


def _sc_grid_body(g_hbm, out_hbm, bufA, bufB, semA, semB):
    wid = lax.axis_index("s") * NC + lax.axis_index("c")
    lo = wid * (NSC * GB) // NW
    hi = (wid + 1) * (NSC * GB) // NW
    oA = jnp.int32(0)
    oB = jnp.int32(0)
    for gi in range(NSC):
        blo = jnp.clip(lo - gi * GB, 0, GB)
        bhi = jnp.clip(hi - gi * GB, 0, GB)

        def gbody(b, o, gi=gi):
            oA, oB = o
            h = b // NBAND
            r = b - h * NBAND
            even = (b % 2) == 0
            src = g_hbm[gi].at[pl.ds(h * W + r * BAND, BAND)]
            dst = out_hbm.at[h, CTC + gi, pl.ds(r * BAND, BAND)]

            @pl.when(jnp.logical_and(even, oA > 0))
            def _():
                _wait_band(bufA, out_hbm, semA)

            @pl.when(jnp.logical_and(jnp.logical_not(even), oB > 0))
            def _():
                _wait_band(bufB, out_hbm, semB)

            @pl.when(even)
            def _():
                pltpu.sync_copy(src, bufA)
                pltpu.async_copy(bufA, dst, semA)

            @pl.when(jnp.logical_not(even))
            def _():
                pltpu.sync_copy(src, bufB)
                pltpu.async_copy(bufB, dst, semB)

            return (jnp.where(even, 1, oA), jnp.where(even, oB, 1))

        oA, oB = lax.fori_loop(blo, bhi, gbody, (oA, oB))

    @pl.when(oA > 0)
    def _():
        _wait_band(bufA, out_hbm, semA)

    @pl.when(oB > 0)
    def _():
        _wait_band(bufB, out_hbm, semB)


def _tc_body(tt_ref, st_ref, par_ref, g0, g1, g2, g3, g4, tbl_ref, _alias,
             out_ref):
    g_refs = (g0, g1, g2, g3, g4)
    tt = tt_ref[...]
    for c in range(EC):
        t0 = tbl_ref[0, c]
        t1 = tbl_ref[1, c]
        t2 = tbl_ref[2, c]
        t3 = tbl_ref[3, c]
        v = jnp.where(tt < 2, jnp.where(tt == 0, t0, t1),
                      jnp.where(tt == 2, t2, t3))
        out_ref[0, c] = v
    out_ref[0, EC] = jnp.broadcast_to(st_ref[...], (W, BB))
    for k in range(P):
        out_ref[0, EC + 1 + k] = jnp.broadcast_to(par_ref[k:k + 1, :], (W, BB))
    for g in range(5):
        out_ref[0, EC + 1 + P + g] = g_refs[g][...]


@jax.jit
def _encode_hy(tt, st, par, grids, tbl):
    mesh = plsc.VectorSubcoreMesh(core_axis_name="c", subcore_axis_name="s")

    def sc_body(g5, g6, g7, g8, g9, out_hbm, bufA, bufB, semA, semB):
        _sc_grid_body((g5, g6, g7, g8, g9), out_hbm, bufA, bufB, semA, semB)

    sc_run = pl.kernel(
        sc_body,
        out_type=jax.ShapeDtypeStruct((H, C, W, B), jnp.float32),
        mesh=mesh,
        scratch_types=[
            pltpu.VMEM((BAND, B), jnp.float32),
            pltpu.VMEM((BAND, B), jnp.float32),
            pltpu.SemaphoreType.DMA,
            pltpu.SemaphoreType.DMA,
        ],
        compiler_params=pltpu.CompilerParams(needs_layout_passes=False),
    )
    part = sc_run(*grids[5:])

    hw_spec = pl.BlockSpec((W, BB), lambda h, j: (h, j))
    tc_run = pl.pallas_call(
        _tc_body,
        grid=(H, NBC),
        in_specs=[
            hw_spec,
            pl.BlockSpec((1, BB), lambda h, j: (0, j)),
            pl.BlockSpec((P, BB), lambda h, j: (0, j)),
        ] + [hw_spec] * 5 + [
            pl.BlockSpec(memory_space=pltpu.SMEM),
            pl.BlockSpec(memory_space=pl.ANY),
        ],
        out_specs=pl.BlockSpec((1, CTC, W, BB), lambda h, j: (h, 0, 0, j)),
        out_shape=jax.ShapeDtypeStruct((H, C, W, B), jnp.float32),
        input_output_aliases={9: 0},
    )
    return tc_run(tt, st, par, *grids[:5], tbl, part)


def kernel(tile_type, normalized_steps, param_list,
           sensor_mask, normalized_unit_counts, normalized_unit_counts_opp,
           normalized_unit_energys_max_grid, normalized_unit_energys_max_grid_opp,
           grid_probability_of_being_an_energy_point_based_on_no_reward,
           grid_max_probability_of_being_an_energy_point_based_on_positive_rewards,
           grid_avg_probability_of_being_an_energy_point_based_on_positive_rewards,
           grid_probability_of_being_energy_point_based_on_relic_positions,
           value_of_sapping_grid, embed_table):
    grids = (sensor_mask, normalized_unit_counts, normalized_unit_counts_opp,
             normalized_unit_energys_max_grid, normalized_unit_energys_max_grid_opp,
             grid_probability_of_being_an_energy_point_based_on_no_reward,
             grid_max_probability_of_being_an_energy_point_based_on_positive_rewards,
             grid_avg_probability_of_being_an_energy_point_based_on_positive_rewards,
             grid_probability_of_being_energy_point_based_on_relic_positions,
             value_of_sapping_grid)
    tt = jnp.transpose(tile_type.astype(jnp.int32), (1, 2, 0)).reshape(HW, B)
    st = normalized_steps.astype(jnp.float32).reshape(1, B)
    par = jnp.transpose(param_list, (1, 0))
    gr = tuple(jnp.transpose(g, (1, 2, 0)).reshape(HW, B) for g in grids)
    out = _encode_hy(tt, st, par, gr, embed_table)
    return jnp.transpose(out, (3, 0, 2, 1))
